# Initial kernel scaffold; baseline (speedup 1.0000x reference)
import jax
import jax.numpy as jnp
from jax.experimental import pallas as pl


def kernel(x, path_lengths, params):
    raise NotImplementedError("write your pallas kernel here")


# speedup vs baseline: 1.1862x; 1.1862x over previous
import functools

import jax
import jax.numpy as jnp
from jax import lax
from jax.experimental import pallas as pl
from jax.experimental.pallas import tpu as pltpu
from jax.experimental.pallas import tpu_sc as plsc

NEXP = 8
TILE = 256
DIN = 768
HMID = 384
DOUT = 256
NWORKERS = 32


def _sc_gather_rows(table, idx, chunk):
    rows, d = table.shape
    b = idx.shape[0]
    per_w = b // NWORKERS
    nchunk = per_w // chunk
    mesh = plsc.VectorSubcoreMesh(core_axis_name="c", subcore_axis_name="s")

    @functools.partial(
        pl.kernel,
        out_type=jax.ShapeDtypeStruct((b, d), table.dtype),
        mesh=mesh,
        scratch_types=[
            pltpu.VMEM((chunk,), jnp.int32),
            pltpu.VMEM((chunk, d), table.dtype),
            pltpu.SemaphoreType.DMA,
        ],
    )
    def gather_kernel(table_hbm, idx_hbm, out_hbm, idx_v, rows_v, sem):
        wid = lax.axis_index("s") * 2 + lax.axis_index("c")
        base = pl.multiple_of(wid * per_w, 8)
        for k in range(nchunk):
            off = base + k * chunk
            pltpu.sync_copy(idx_hbm.at[pl.ds(off, chunk)], idx_v)
            pltpu.async_copy(table_hbm.at[idx_v], rows_v, sem).wait()
            pltpu.sync_copy(rows_v, out_hbm.at[pl.ds(off, chunk)])

    return gather_kernel(table, idx)


def _tc_mlp_body(e_map, x_ref, w0, w1, w2, w3, w4, b0, b1, b2, b3, b4, o_ref):
    h = x_ref[...]
    h = jnp.maximum(jnp.dot(h, w0[0], preferred_element_type=jnp.float32) + b0[0], 0.0)
    h = jnp.maximum(jnp.dot(h, w1[0], preferred_element_type=jnp.float32) + b1[0], 0.0)
    h = jnp.maximum(jnp.dot(h, w2[0], preferred_element_type=jnp.float32) + b2[0], 0.0)
    h = jnp.maximum(jnp.dot(h, w3[0], preferred_element_type=jnp.float32) + b3[0], 0.0)
    o_ref[...] = jnp.dot(h, w4[0], preferred_element_type=jnp.float32) + b4[0]


def _tc_grouped_mlp(x_sorted, e_of_tile, ws, bs):
    np_rows = x_sorted.shape[0]
    nt = np_rows // TILE
    w0, w1, w2, w3, w4 = ws
    b0, b1, b2, b3, b4 = bs

    def wspec(shape):
        return pl.BlockSpec((1,) + shape, lambda t, e: (e[t], 0, 0))

    grid_spec = pltpu.PrefetchScalarGridSpec(
        num_scalar_prefetch=1,
        grid=(nt,),
        in_specs=[
            pl.BlockSpec((TILE, DIN), lambda t, e: (t, 0)),
            wspec((DIN, HMID)),
            wspec((HMID, HMID)),
            wspec((HMID, HMID)),
            wspec((HMID, HMID)),
            wspec((HMID, DOUT)),
            wspec((1, HMID)),
            wspec((1, HMID)),
            wspec((1, HMID)),
            wspec((1, HMID)),
            wspec((1, DOUT)),
        ],
        out_specs=pl.BlockSpec((TILE, DOUT), lambda t, e: (t, 0)),
    )
    return pl.pallas_call(
        _tc_mlp_body,
        grid_spec=grid_spec,
        out_shape=jax.ShapeDtypeStruct((np_rows, DOUT), jnp.float32),
    )(e_of_tile, x_sorted, w0, w1, w2, w3, w4, b0, b1, b2, b3, b4)


def _pack_params(params):
    eye = jnp.eye(HMID, dtype=jnp.float32)
    zb = jnp.zeros((HMID,), dtype=jnp.float32)

    def padw(w, rin, rout):
        return jnp.zeros((rin, rout), jnp.float32).at[: w.shape[0], : w.shape[1]].set(w)

    def padb(b, r):
        return jnp.zeros((r,), jnp.float32).at[: b.shape[0]].set(b)

    slot_w = [[] for _ in range(5)]
    slot_b = [[] for _ in range(5)]
    for mlp in params:
        n = len(mlp)
        if n == 3:
            order = [0, 1, None, None, 2]
        elif n == 4:
            order = [0, 1, 2, None, 3]
        else:
            order = [0, 1, 2, 3, 4]
        for s, li in enumerate(order):
            rin = DIN if s == 0 else HMID
            rout = DOUT if s == 4 else HMID
            if li is None:
                slot_w[s].append(eye)
                slot_b[s].append(zb)
            else:
                w, b = mlp[li]
                slot_w[s].append(padw(w, rin, rout))
                slot_b[s].append(padb(b, rout))
    ws = tuple(jnp.stack(slot_w[s]) for s in range(5))
    bs = tuple(jnp.stack(slot_b[s])[:, None, :] for s in range(5))
    return ws, bs


def kernel(x, path_lengths, params):
    batch, nodes, _ = x.shape
    n = batch * nodes
    np_rows = n + NEXP * TILE
    nt = np_rows // TILE

    x_flat = x.reshape(n, DIN)
    plf = jnp.clip(path_lengths.reshape(n), 0, NEXP - 1).astype(jnp.int32)

    onehot = (plf[:, None] == jnp.arange(NEXP, dtype=jnp.int32)[None, :]).astype(jnp.int32)
    incl = jnp.cumsum(onehot, axis=0)
    counts = incl[-1]
    rank = jnp.take_along_axis(incl, plf[:, None], axis=1)[:, 0] - 1
    tiles_e = (counts + TILE - 1) // TILE
    tile_end = jnp.cumsum(tiles_e)
    row_off = (tile_end - tiles_e) * TILE
    pos = (row_off[plf] + rank).astype(jnp.int32)

    src_row = jnp.zeros((np_rows,), jnp.int32).at[pos].set(
        jnp.arange(n, dtype=jnp.int32))
    e_of_tile = jnp.minimum(
        jnp.searchsorted(tile_end, jnp.arange(nt, dtype=jnp.int32), side="right"),
        NEXP - 1,
    ).astype(jnp.int32)

    ws, bs = _pack_params(params)

    x_sorted = _sc_gather_rows(x_flat, src_row, chunk=64)
    out_sorted = _tc_grouped_mlp(x_sorted, e_of_tile, ws, bs)
    out = _sc_gather_rows(out_sorted, pos, chunk=128)
    return out.reshape(batch, nodes, DOUT)


# speedup vs baseline: 1.2465x; 1.0509x over previous
import functools

import jax
import jax.numpy as jnp
from jax import lax
from jax.experimental import pallas as pl
from jax.experimental.pallas import tpu as pltpu
from jax.experimental.pallas import tpu_sc as plsc

NEXP = 8
TILE = 256
DIN = 768
HMID = 384
DOUT = 256
NWORKERS = 32


def _sc_gather_rows(table, idx, chunk):
    rows, d = table.shape
    b = idx.shape[0]
    per_w = b // NWORKERS
    nchunk = per_w // chunk
    mesh = plsc.VectorSubcoreMesh(core_axis_name="c", subcore_axis_name="s")

    @functools.partial(
        pl.kernel,
        out_type=jax.ShapeDtypeStruct((b, d), table.dtype),
        mesh=mesh,
        scratch_types=[
            pltpu.VMEM((per_w,), jnp.int32),
            pltpu.VMEM((2, chunk, d), table.dtype),
            pltpu.SemaphoreType.DMA((2,)),
            pltpu.SemaphoreType.DMA((2,)),
        ],
    )
    def gather_kernel(table_hbm, idx_hbm, out_hbm, idx_v, rows_v, gsem, osem):
        wid = lax.axis_index("s") * 2 + lax.axis_index("c")
        base = pl.multiple_of(wid * per_w, 8)
        pltpu.sync_copy(idx_hbm.at[pl.ds(base, per_w)], idx_v)

        def start_gather(k):
            slot = k % 2
            return pltpu.async_copy(
                table_hbm.at[idx_v.at[pl.ds(k * chunk, chunk)]],
                rows_v.at[slot], gsem.at[slot])

        def start_out(k):
            slot = k % 2
            return pltpu.async_copy(
                rows_v.at[slot], out_hbm.at[pl.ds(base + k * chunk, chunk)],
                osem.at[slot])

        g = [None, None]
        o = [None, None]
        g[0] = start_gather(0)
        for k in range(nchunk):
            slot = k % 2
            nxt = (k + 1) % 2
            if k + 1 < nchunk:
                if o[nxt] is not None:
                    o[nxt].wait()
                g[nxt] = start_gather(k + 1)
            g[slot].wait()
            o[slot] = start_out(k)
        o[(nchunk - 1) % 2].wait()
        if nchunk >= 2:
            o[nchunk % 2].wait()

    return gather_kernel(table, idx)


def _tc_mlp_body(e_map, x_ref, w0, w1, w2, w3, w4, b0, b1, b2, b3, b4, o_ref):
    h = x_ref[...].astype(jnp.bfloat16)

    def lin(h, w, b):
        return jnp.dot(h, w[0].astype(jnp.bfloat16),
                       preferred_element_type=jnp.float32) + b[0]

    h = jnp.maximum(lin(h, w0, b0), 0.0).astype(jnp.bfloat16)
    h = jnp.maximum(lin(h, w1, b1), 0.0).astype(jnp.bfloat16)
    h = jnp.maximum(lin(h, w2, b2), 0.0).astype(jnp.bfloat16)
    h = jnp.maximum(lin(h, w3, b3), 0.0).astype(jnp.bfloat16)
    o_ref[...] = lin(h, w4, b4)


def _tc_grouped_mlp(x_sorted, e_of_tile, ws, bs):
    np_rows = x_sorted.shape[0]
    nt = np_rows // TILE
    w0, w1, w2, w3, w4 = ws
    b0, b1, b2, b3, b4 = bs

    def wspec(shape):
        return pl.BlockSpec((1,) + shape, lambda t, e: (e[t], 0, 0))

    grid_spec = pltpu.PrefetchScalarGridSpec(
        num_scalar_prefetch=1,
        grid=(nt,),
        in_specs=[
            pl.BlockSpec((TILE, DIN), lambda t, e: (t, 0)),
            wspec((DIN, HMID)),
            wspec((HMID, HMID)),
            wspec((HMID, HMID)),
            wspec((HMID, HMID)),
            wspec((HMID, DOUT)),
            wspec((1, HMID)),
            wspec((1, HMID)),
            wspec((1, HMID)),
            wspec((1, HMID)),
            wspec((1, DOUT)),
        ],
        out_specs=pl.BlockSpec((TILE, DOUT), lambda t, e: (t, 0)),
    )
    return pl.pallas_call(
        _tc_mlp_body,
        grid_spec=grid_spec,
        out_shape=jax.ShapeDtypeStruct((np_rows, DOUT), jnp.float32),
    )(e_of_tile, x_sorted, w0, w1, w2, w3, w4, b0, b1, b2, b3, b4)


def _pack_params(params):
    eye = jnp.eye(HMID, dtype=jnp.float32)
    zb = jnp.zeros((HMID,), dtype=jnp.float32)

    def padw(w, rin, rout):
        return jnp.zeros((rin, rout), jnp.float32).at[: w.shape[0], : w.shape[1]].set(w)

    def padb(b, r):
        return jnp.zeros((r,), jnp.float32).at[: b.shape[0]].set(b)

    slot_w = [[] for _ in range(5)]
    slot_b = [[] for _ in range(5)]
    for mlp in params:
        n = len(mlp)
        if n == 3:
            order = [0, 1, None, None, 2]
        elif n == 4:
            order = [0, 1, 2, None, 3]
        else:
            order = [0, 1, 2, 3, 4]
        for s, li in enumerate(order):
            rin = DIN if s == 0 else HMID
            rout = DOUT if s == 4 else HMID
            if li is None:
                slot_w[s].append(eye)
                slot_b[s].append(zb)
            else:
                w, b = mlp[li]
                slot_w[s].append(padw(w, rin, rout))
                slot_b[s].append(padb(b, rout))
    ws = tuple(jnp.stack(slot_w[s]) for s in range(5))
    bs = tuple(jnp.stack(slot_b[s])[:, None, :] for s in range(5))
    return ws, bs


def kernel(x, path_lengths, params):
    batch, nodes, _ = x.shape
    n = batch * nodes
    np_rows = n + NEXP * TILE
    nt = np_rows // TILE

    x_flat = x.reshape(n, DIN)
    plf = jnp.clip(path_lengths.reshape(n), 0, NEXP - 1).astype(jnp.int32)

    onehot = (plf[:, None] == jnp.arange(NEXP, dtype=jnp.int32)[None, :]).astype(jnp.int32)
    incl = jnp.cumsum(onehot, axis=0)
    counts = incl[-1]
    rank = jnp.take_along_axis(incl, plf[:, None], axis=1)[:, 0] - 1
    tiles_e = (counts + TILE - 1) // TILE
    tile_end = jnp.cumsum(tiles_e)
    row_off = (tile_end - tiles_e) * TILE
    pos = (row_off[plf] + rank).astype(jnp.int32)

    src_row = jnp.zeros((np_rows,), jnp.int32).at[pos].set(
        jnp.arange(n, dtype=jnp.int32))
    e_of_tile = jnp.minimum(
        jnp.searchsorted(tile_end, jnp.arange(nt, dtype=jnp.int32), side="right"),
        NEXP - 1,
    ).astype(jnp.int32)

    ws, bs = _pack_params(params)

    x_sorted = _sc_gather_rows(x_flat, src_row, chunk=64)
    out_sorted = _tc_grouped_mlp(x_sorted, e_of_tile, ws, bs)
    out = _sc_gather_rows(out_sorted, pos, chunk=128)
    return out.reshape(batch, nodes, DOUT)


# speedup vs baseline: 2.5030x; 2.0080x over previous
import functools

import jax
import jax.numpy as jnp
from jax import lax
from jax.experimental import pallas as pl
from jax.experimental.pallas import tpu as pltpu
from jax.experimental.pallas import tpu_sc as plsc

NEXP = 8
TILE = 256
DIN = 768
HMID = 384
DOUT = 256
NWORKERS = 32


def _sc_gather_rows(table, idx, chunk):
    rows, d = table.shape
    b = idx.shape[0]
    per_w = b // NWORKERS
    nchunk = per_w // chunk
    mesh = plsc.VectorSubcoreMesh(core_axis_name="c", subcore_axis_name="s")

    @functools.partial(
        pl.kernel,
        out_type=jax.ShapeDtypeStruct((b, d), table.dtype),
        mesh=mesh,
        scratch_types=[
            pltpu.VMEM((per_w,), jnp.int32),
            pltpu.VMEM((2, chunk, d), table.dtype),
            pltpu.SemaphoreType.DMA((2,)),
            pltpu.SemaphoreType.DMA((2,)),
        ],
    )
    def gather_kernel(table_hbm, idx_hbm, out_hbm, idx_v, rows_v, gsem, osem):
        wid = lax.axis_index("s") * 2 + lax.axis_index("c")
        base = pl.multiple_of(wid * per_w, 8)
        pltpu.sync_copy(idx_hbm.at[pl.ds(base, per_w)], idx_v)

        def start_gather(k):
            slot = k % 2
            return pltpu.async_copy(
                table_hbm.at[idx_v.at[pl.ds(k * chunk, chunk)]],
                rows_v.at[slot], gsem.at[slot])

        def start_out(k):
            slot = k % 2
            return pltpu.async_copy(
                rows_v.at[slot], out_hbm.at[pl.ds(base + k * chunk, chunk)],
                osem.at[slot])

        g = [None, None]
        o = [None, None]
        g[0] = start_gather(0)
        for k in range(nchunk):
            slot = k % 2
            nxt = (k + 1) % 2
            if k + 1 < nchunk:
                if o[nxt] is not None:
                    o[nxt].wait()
                g[nxt] = start_gather(k + 1)
            g[slot].wait()
            o[slot] = start_out(k)
        o[(nchunk - 1) % 2].wait()
        if nchunk >= 2:
            o[nchunk % 2].wait()

    return gather_kernel(table, idx)


def _sc_scatter_rows(src, pos, out_rows, chunk):
    n, d = src.shape
    per_w = n // NWORKERS
    nchunk = per_w // chunk
    pos3d = pos.reshape(NWORKERS, nchunk, chunk)
    mesh = plsc.VectorSubcoreMesh(core_axis_name="c", subcore_axis_name="s")

    @functools.partial(
        pl.kernel,
        out_type=jax.ShapeDtypeStruct((out_rows, d), src.dtype),
        mesh=mesh,
        scratch_types=[
            pltpu.VMEM((nchunk, chunk), jnp.int32),
            pltpu.VMEM((2, chunk, d), src.dtype),
            pltpu.SemaphoreType.DMA((2,)),
            pltpu.SemaphoreType.DMA((2,)),
        ],
    )
    def scatter_kernel(src_hbm, pos_hbm, out_hbm, idx_v, rows_v, isem, osem):
        wid = lax.axis_index("s") * 2 + lax.axis_index("c")
        base = pl.multiple_of(wid * per_w, 8)
        pltpu.sync_copy(pos_hbm.at[wid], idx_v)

        def start_in(k):
            slot = k % 2
            return pltpu.async_copy(
                src_hbm.at[pl.ds(base + k * chunk, chunk)],
                rows_v.at[slot], isem.at[slot])

        def start_out(k):
            slot = k % 2
            return pltpu.async_copy(
                rows_v.at[slot], out_hbm.at[idx_v.at[k]], osem.at[slot])

        g = [None, None]
        o = [None, None]
        g[0] = start_in(0)
        for k in range(nchunk):
            slot = k % 2
            nxt = (k + 1) % 2
            if k + 1 < nchunk:
                if o[nxt] is not None:
                    o[nxt].wait()
                g[nxt] = start_in(k + 1)
            g[slot].wait()
            o[slot] = start_out(k)
        o[(nchunk - 1) % 2].wait()
        if nchunk >= 2:
            o[nchunk % 2].wait()

    return scatter_kernel(src, pos3d)


def _tc_mlp_body(e_map, x_ref, w0, w1, w2, w3, w4, b0, b1, b2, b3, b4, o_ref):
    e = e_map[pl.program_id(0)]
    xb = x_ref[...].astype(jnp.bfloat16)

    def lin(h, wref, bref, din, dout):
        w = wref[0, :din, :dout].astype(jnp.bfloat16)
        return jnp.dot(h, w, preferred_element_type=jnp.float32) + bref[0, :, :dout]

    def rl(h):
        return jnp.maximum(h, 0.0).astype(jnp.bfloat16)

    @pl.when(e < 2)
    def _():
        h = rl(lin(xb, w0, b0, 768, 384))
        h = rl(lin(h, w1, b1, 384, 384))
        o_ref[...] = lin(h, w4, b4, 384, 256)

    @pl.when((e >= 2) & (e < 4))
    def _():
        h = rl(lin(xb, w0, b0, 768, 256))
        h = rl(lin(h, w1, b1, 256, 256))
        h = rl(lin(h, w2, b2, 256, 256))
        o_ref[...] = lin(h, w4, b4, 256, 256)

    @pl.when(e >= 4)
    def _():
        h = rl(lin(xb, w0, b0, 768, 256))
        h = rl(lin(h, w1, b1, 256, 256))
        h = rl(lin(h, w2, b2, 256, 256))
        h = rl(lin(h, w3, b3, 256, 256))
        o_ref[...] = lin(h, w4, b4, 256, 256)


def _tc_grouped_mlp(x_sorted, e_of_tile, ws, bs):
    np_rows = x_sorted.shape[0]
    nt = np_rows // TILE
    w0, w1, w2, w3, w4 = ws
    b0, b1, b2, b3, b4 = bs

    def wspec(shape):
        return pl.BlockSpec((1,) + shape, lambda t, e: (e[t], 0, 0))

    grid_spec = pltpu.PrefetchScalarGridSpec(
        num_scalar_prefetch=1,
        grid=(nt,),
        in_specs=[
            pl.BlockSpec((TILE, DIN), lambda t, e: (t, 0)),
            wspec((DIN, HMID)),
            wspec((HMID, HMID)),
            wspec((HMID, HMID)),
            wspec((HMID, HMID)),
            wspec((HMID, DOUT)),
            wspec((1, HMID)),
            wspec((1, HMID)),
            wspec((1, HMID)),
            wspec((1, HMID)),
            wspec((1, DOUT)),
        ],
        out_specs=pl.BlockSpec((TILE, DOUT), lambda t, e: (t, 0)),
    )
    return pl.pallas_call(
        _tc_mlp_body,
        grid_spec=grid_spec,
        out_shape=jax.ShapeDtypeStruct((np_rows, DOUT), jnp.float32),
    )(e_of_tile, x_sorted, w0, w1, w2, w3, w4, b0, b1, b2, b3, b4)


def _pack_params(params):
    eye = jnp.eye(HMID, dtype=jnp.float32)
    zb = jnp.zeros((HMID,), dtype=jnp.float32)

    def padw(w, rin, rout):
        return jnp.zeros((rin, rout), jnp.float32).at[: w.shape[0], : w.shape[1]].set(w)

    def padb(b, r):
        return jnp.zeros((r,), jnp.float32).at[: b.shape[0]].set(b)

    slot_w = [[] for _ in range(5)]
    slot_b = [[] for _ in range(5)]
    for mlp in params:
        n = len(mlp)
        if n == 3:
            order = [0, 1, None, None, 2]
        elif n == 4:
            order = [0, 1, 2, None, 3]
        else:
            order = [0, 1, 2, 3, 4]
        for s, li in enumerate(order):
            rin = DIN if s == 0 else HMID
            rout = DOUT if s == 4 else HMID
            if li is None:
                slot_w[s].append(eye)
                slot_b[s].append(zb)
            else:
                w, b = mlp[li]
                slot_w[s].append(padw(w, rin, rout))
                slot_b[s].append(padb(b, rout))
    ws = tuple(jnp.stack(slot_w[s]) for s in range(5))
    bs = tuple(jnp.stack(slot_b[s])[:, None, :] for s in range(5))
    return ws, bs


def kernel(x, path_lengths, params):
    batch, nodes, _ = x.shape
    n = batch * nodes
    np_rows = n + NEXP * TILE
    nt = np_rows // TILE

    x_flat = x.reshape(n, DIN)
    plf = jnp.clip(path_lengths.reshape(n), 0, NEXP - 1).astype(jnp.int32)

    onehot = (plf[:, None] == jnp.arange(NEXP, dtype=jnp.int32)[None, :]).astype(jnp.int32)
    incl = jnp.cumsum(onehot, axis=0)
    counts = incl[-1]
    tiles_e = (counts + TILE - 1) // TILE
    tile_end = jnp.cumsum(tiles_e)
    row_off = (tile_end - tiles_e) * TILE
    pos = (jnp.sum((row_off[None, :] + incl) * onehot, axis=1) - 1).astype(jnp.int32)
    e_of_tile = jnp.minimum(
        jnp.searchsorted(tile_end, jnp.arange(nt, dtype=jnp.int32), side="right"),
        NEXP - 1,
    ).astype(jnp.int32)

    ws, bs = _pack_params(params)

    x_sorted = _sc_scatter_rows(x_flat, pos, np_rows, chunk=64)
    out_sorted = _tc_grouped_mlp(x_sorted, e_of_tile, ws, bs)
    out = _sc_gather_rows(out_sorted, pos, chunk=128)
    return out.reshape(batch, nodes, DOUT)


# speedup vs baseline: 3.0430x; 1.2157x over previous
import functools

import jax
import jax.numpy as jnp
from jax import lax
from jax.experimental import pallas as pl
from jax.experimental.pallas import tpu as pltpu
from jax.experimental.pallas import tpu_sc as plsc

NEXP = 8
TILE = 512
DIN = 768
HMID = 384
DOUT = 256
NWORKERS = 32


def _sc_gather_rows(table, idx, chunk):
    rows, d = table.shape
    b = idx.shape[0]
    per_w = b // NWORKERS
    nchunk = per_w // chunk
    mesh = plsc.VectorSubcoreMesh(core_axis_name="c", subcore_axis_name="s")

    @functools.partial(
        pl.kernel,
        out_type=jax.ShapeDtypeStruct((b, d), table.dtype),
        mesh=mesh,
        scratch_types=[
            pltpu.VMEM((per_w,), jnp.int32),
            pltpu.VMEM((2, chunk, d), table.dtype),
            pltpu.SemaphoreType.DMA((2,)),
            pltpu.SemaphoreType.DMA((2,)),
        ],
    )
    def gather_kernel(table_hbm, idx_hbm, out_hbm, idx_v, rows_v, gsem, osem):
        wid = lax.axis_index("s") * 2 + lax.axis_index("c")
        base = pl.multiple_of(wid * per_w, 8)
        pltpu.sync_copy(idx_hbm.at[pl.ds(base, per_w)], idx_v)

        def start_gather(k):
            slot = k % 2
            return pltpu.async_copy(
                table_hbm.at[idx_v.at[pl.ds(k * chunk, chunk)]],
                rows_v.at[slot], gsem.at[slot])

        def start_out(k):
            slot = k % 2
            return pltpu.async_copy(
                rows_v.at[slot], out_hbm.at[pl.ds(base + k * chunk, chunk)],
                osem.at[slot])

        g = [None, None]
        o = [None, None]
        g[0] = start_gather(0)
        for k in range(nchunk):
            slot = k % 2
            nxt = (k + 1) % 2
            if k + 1 < nchunk:
                if o[nxt] is not None:
                    o[nxt].wait()
                g[nxt] = start_gather(k + 1)
            g[slot].wait()
            o[slot] = start_out(k)
        o[(nchunk - 1) % 2].wait()
        if nchunk >= 2:
            o[nchunk % 2].wait()

    return gather_kernel(table, idx)


def _sc_scatter_rows(src, pos, out_rows, chunk):
    n, d = src.shape
    per_w = n // NWORKERS
    nchunk = per_w // chunk
    pos3d = pos.reshape(NWORKERS, nchunk, chunk)
    mesh = plsc.VectorSubcoreMesh(core_axis_name="c", subcore_axis_name="s")

    @functools.partial(
        pl.kernel,
        out_type=jax.ShapeDtypeStruct((out_rows, d), src.dtype),
        mesh=mesh,
        scratch_types=[
            pltpu.VMEM((nchunk, chunk), jnp.int32),
            pltpu.VMEM((2, chunk, d), src.dtype),
            pltpu.SemaphoreType.DMA((2,)),
            pltpu.SemaphoreType.DMA((2,)),
        ],
    )
    def scatter_kernel(src_hbm, pos_hbm, out_hbm, idx_v, rows_v, isem, osem):
        wid = lax.axis_index("s") * 2 + lax.axis_index("c")
        base = pl.multiple_of(wid * per_w, 8)
        pltpu.sync_copy(pos_hbm.at[wid], idx_v)

        def start_in(k):
            slot = k % 2
            return pltpu.async_copy(
                src_hbm.at[pl.ds(base + k * chunk, chunk)],
                rows_v.at[slot], isem.at[slot])

        def start_out(k):
            slot = k % 2
            return pltpu.async_copy(
                rows_v.at[slot], out_hbm.at[idx_v.at[k]], osem.at[slot])

        g = [None, None]
        o = [None, None]
        g[0] = start_in(0)
        for k in range(nchunk):
            slot = k % 2
            nxt = (k + 1) % 2
            if k + 1 < nchunk:
                if o[nxt] is not None:
                    o[nxt].wait()
                g[nxt] = start_in(k + 1)
            g[slot].wait()
            o[slot] = start_out(k)
        o[(nchunk - 1) % 2].wait()
        if nchunk >= 2:
            o[nchunk % 2].wait()

    return scatter_kernel(src, pos3d)


def _tc_mlp_body(e_map, x_ref, w0, w1, w2, w3, w4, b0, b1, b2, b3, b4, o_ref):
    e = e_map[pl.program_id(0)]
    xb = x_ref[...].astype(jnp.bfloat16)

    def lin(h, wref, bref, din, dout):
        w = wref[0, :din, :dout].astype(jnp.bfloat16)
        return jnp.dot(h, w, preferred_element_type=jnp.float32) + bref[0, :, :dout]

    def rl(h):
        return jnp.maximum(h, 0.0).astype(jnp.bfloat16)

    @pl.when(e < 2)
    def _():
        h = rl(lin(xb, w0, b0, 768, 384))
        h = rl(lin(h, w1, b1, 384, 384))
        o_ref[...] = lin(h, w4, b4, 384, 256)

    @pl.when((e >= 2) & (e < 4))
    def _():
        h = rl(lin(xb, w0, b0, 768, 256))
        h = rl(lin(h, w1, b1, 256, 256))
        h = rl(lin(h, w2, b2, 256, 256))
        o_ref[...] = lin(h, w4, b4, 256, 256)

    @pl.when(e >= 4)
    def _():
        h = rl(lin(xb, w0, b0, 768, 256))
        h = rl(lin(h, w1, b1, 256, 256))
        h = rl(lin(h, w2, b2, 256, 256))
        h = rl(lin(h, w3, b3, 256, 256))
        o_ref[...] = lin(h, w4, b4, 256, 256)


def _tc_grouped_mlp(x_sorted, e_of_tile, ws, bs):
    np_rows = x_sorted.shape[0]
    nt = np_rows // TILE
    w0, w1, w2, w3, w4 = ws
    b0, b1, b2, b3, b4 = bs

    def wspec(shape):
        return pl.BlockSpec((1,) + shape, lambda t, e: (e[t], 0, 0))

    grid_spec = pltpu.PrefetchScalarGridSpec(
        num_scalar_prefetch=1,
        grid=(nt,),
        in_specs=[
            pl.BlockSpec((TILE, DIN), lambda t, e: (t, 0)),
            wspec((DIN, HMID)),
            wspec((HMID, HMID)),
            wspec((HMID, HMID)),
            wspec((HMID, HMID)),
            wspec((HMID, DOUT)),
            wspec((1, HMID)),
            wspec((1, HMID)),
            wspec((1, HMID)),
            wspec((1, HMID)),
            wspec((1, DOUT)),
        ],
        out_specs=pl.BlockSpec((TILE, DOUT), lambda t, e: (t, 0)),
    )
    return pl.pallas_call(
        _tc_mlp_body,
        grid_spec=grid_spec,
        out_shape=jax.ShapeDtypeStruct((np_rows, DOUT), jnp.float32),
    )(e_of_tile, x_sorted, w0, w1, w2, w3, w4, b0, b1, b2, b3, b4)


def _pack_params(params):
    eye = jnp.eye(HMID, dtype=jnp.float32)
    zb = jnp.zeros((HMID,), dtype=jnp.float32)

    def padw(w, rin, rout):
        return jnp.zeros((rin, rout), jnp.float32).at[: w.shape[0], : w.shape[1]].set(w)

    def padb(b, r):
        return jnp.zeros((r,), jnp.float32).at[: b.shape[0]].set(b)

    slot_w = [[] for _ in range(5)]
    slot_b = [[] for _ in range(5)]
    for mlp in params:
        n = len(mlp)
        if n == 3:
            order = [0, 1, None, None, 2]
        elif n == 4:
            order = [0, 1, 2, None, 3]
        else:
            order = [0, 1, 2, 3, 4]
        for s, li in enumerate(order):
            rin = DIN if s == 0 else HMID
            rout = DOUT if s == 4 else HMID
            if li is None:
                slot_w[s].append(eye)
                slot_b[s].append(zb)
            else:
                w, b = mlp[li]
                slot_w[s].append(padw(w, rin, rout))
                slot_b[s].append(padb(b, rout))
    ws = tuple(jnp.stack(slot_w[s]) for s in range(5))
    bs = tuple(jnp.stack(slot_b[s])[:, None, :] for s in range(5))
    return ws, bs


def kernel(x, path_lengths, params):
    batch, nodes, _ = x.shape
    n = batch * nodes
    np_rows = n + NEXP * TILE
    nt = np_rows // TILE

    x_flat = x.reshape(n, DIN)
    plf = jnp.clip(path_lengths.reshape(n), 0, NEXP - 1).astype(jnp.int32)

    onehot = (plf[:, None] == jnp.arange(NEXP, dtype=jnp.int32)[None, :]).astype(jnp.int32)
    incl = jnp.cumsum(onehot, axis=0)
    counts = incl[-1]
    tiles_e = (counts + TILE - 1) // TILE
    tile_end = jnp.cumsum(tiles_e)
    row_off = (tile_end - tiles_e) * TILE
    pos = (jnp.sum((row_off[None, :] + incl) * onehot, axis=1) - 1).astype(jnp.int32)
    e_of_tile = jnp.minimum(
        jnp.searchsorted(tile_end, jnp.arange(nt, dtype=jnp.int32), side="right"),
        NEXP - 1,
    ).astype(jnp.int32)

    ws, bs = _pack_params(params)

    x_sorted = _sc_scatter_rows(x_flat, pos, np_rows, chunk=64)
    out_sorted = _tc_grouped_mlp(x_sorted, e_of_tile, ws, bs)
    out = _sc_gather_rows(out_sorted, pos, chunk=128)
    return out.reshape(batch, nodes, DOUT)
